# Initial kernel scaffold; baseline (speedup 1.0000x reference)
import jax
import jax.numpy as jnp
from jax.experimental import pallas as pl


def kernel(input, level_weight, features_weight):
    raise NotImplementedError("write your pallas kernel here")


# speedup vs baseline: 1.4339x; 1.4339x over previous
import functools

import jax
import jax.numpy as jnp
from jax.experimental import pallas as pl
from jax.experimental.pallas import tpu as pltpu

_MAX_VAL = 52000.0
_MIN_VAL = -53000.0
_NUM_LEVELS = 1000
_LEVELS_PAD = 1024
_N = 4
_C = 24
_T = 256
_D = 4096


def _roll_lanes(x, shift):
    return jnp.concatenate([x[:, -shift:], x[:, :-shift]], axis=1)


def _body(inT_ref, L_ref, F_ref, out_ref, smp_ref):
    c = pl.program_id(0)
    xcol = inT_ref[0]
    y = (xcol - _MIN_VAL) / (_MAX_VAL - _MIN_VAL) * (_NUM_LEVELS - 1)
    idx = jnp.clip(jnp.round(y), 0, _NUM_LEVELS - 1).astype(jnp.int32)
    lvl = jax.lax.broadcasted_iota(jnp.int32, (_T, _LEVELS_PAD), 1)
    oh = (idx == lvl).astype(jnp.bfloat16)
    g = jnp.dot(oh, L_ref[...], preferred_element_type=jnp.float32)
    prod = g * F_ref[...].astype(jnp.float32)
    s = jnp.sum(prod, axis=0, keepdims=True)
    smp_ref[pl.ds(c, 1), :] = jnp.where(s > 0, 1.0, -1.0)

    @pl.when(c == _C - 1)
    def _():
        qa = smp_ref[...]
        r3 = _roll_lanes(qa, 3)
        r2 = _roll_lanes(qa, 2)
        r1 = _roll_lanes(qa, 1)
        w = (r3[0 : _C - 3] * r2[1 : _C - 2]) * (r1[2 : _C - 1] * qa[3:_C])
        s2 = jnp.sum(w, axis=0, keepdims=True)
        out_ref[...] = jnp.where(s2 > 0, 1.0, -1.0)


@jax.jit
def kernel(input, level_weight, features_weight):
    inT = input[:, :, None]
    Lp = jnp.pad(level_weight, ((0, _LEVELS_PAD - _NUM_LEVELS), (0, 0)))
    Lp = Lp.astype(jnp.bfloat16)
    F = features_weight.astype(jnp.bfloat16)
    out = pl.pallas_call(
        _body,
        grid=(_C,),
        in_specs=[
            pl.BlockSpec((1, _T, 1), lambda c: (c, 0, 0)),
            pl.BlockSpec((_LEVELS_PAD, _D), lambda c: (0, 0)),
            pl.BlockSpec((_T, _D), lambda c: (0, 0)),
        ],
        out_specs=pl.BlockSpec((1, _D), lambda c: (0, 0)),
        out_shape=jax.ShapeDtypeStruct((1, _D), jnp.float32),
        scratch_shapes=[pltpu.VMEM((_C, _D), jnp.float32)],
    )(inT, Lp, F)
    return out
